# baseline (speedup 1.0000x reference)
import jax
import jax.numpy as jnp
from jax.experimental import pallas as pl
from jax.experimental.pallas import tpu as pltpu

_R, _L = 8, 128
_CHUNKS = 5
_CK = _R * _L
_C = _CHUNKS * _L
_NP = _CHUNKS * _CK
_NMS_THRESH = 0.3
_SCORE_THRESH = 0.05


def _nms_body(bt_ref, bs_ref, out_ref, keep_ref):
    S = bt_ref[40:48, :]
    keep_ref[...] = jnp.where(S > _SCORE_THRESH, 1.0, 0.0)

    IDXC = (jax.lax.broadcasted_iota(jnp.int32, (_R, _L), 0) * _L
            + jax.lax.broadcasted_iota(jnp.int32, (_R, _L), 1)
            ).astype(jnp.float32)

    for k in range(_CHUNKS):
        sl = slice(k * _L, (k + 1) * _L)
        tl = slice((k + 1) * _L, _C)
        Yc1 = bt_ref[0:8, sl]
        Xc1 = bt_ref[8:16, sl]
        Yc2 = bt_ref[16:24, sl]
        Xc2 = bt_ref[24:32, sl]
        Ac = bt_ref[32:40, sl]
        if k < _CHUNKS - 1:
            Yt1 = bt_ref[0:8, tl]
            Xt1 = bt_ref[8:16, tl]
            Yt2 = bt_ref[16:24, tl]
            Xt2 = bt_ref[24:32, tl]
            At = bt_ref[32:40, tl]

        keepc0 = keep_ref[:, sl]
        cur0 = jnp.min(jnp.where(keepc0 > 0.0, IDXC, float(_CK)))

        def cond(carry):
            return carry[0] < float(_CK)

        def body(carry, k=k):
            cur, keepc = carry
            ig = (k * _CK + cur.astype(jnp.int32)) * 5
            y1i = bs_ref[ig]
            x1i = bs_ref[ig + 1]
            y2i = bs_ref[ig + 2]
            x2i = bs_ref[ig + 3]
            ai = bs_ref[ig + 4]
            h = jnp.maximum(jnp.minimum(Yc2, y2i) - jnp.maximum(Yc1, y1i), 0.0)
            w = jnp.maximum(jnp.minimum(Xc2, x2i) - jnp.maximum(Xc1, x1i), 0.0)
            inter = h * w
            iou = inter / jnp.maximum((ai + Ac) - inter, 1e-9)
            supc = (iou > _NMS_THRESH) & (IDXC > cur)
            keepc_new = jnp.where(supc, 0.0, keepc)
            if k < _CHUNKS - 1:
                ht = jnp.maximum(
                    jnp.minimum(Yt2, y2i) - jnp.maximum(Yt1, y1i), 0.0)
                wt = jnp.maximum(
                    jnp.minimum(Xt2, x2i) - jnp.maximum(Xt1, x1i), 0.0)
                intert = ht * wt
                iout = intert / jnp.maximum((ai + At) - intert, 1e-9)
                keep_ref[:, tl] = jnp.where(
                    iout > _NMS_THRESH, 0.0, keep_ref[:, tl])
            nxt = jnp.min(
                jnp.where((keepc_new > 0.0) & (IDXC > cur), IDXC, float(_CK)))
            return nxt, keepc_new

        _, keepc_fin = jax.lax.while_loop(cond, body, (cur0, keepc0))
        keep_ref[:, sl] = keepc_fin

    out_ref[...] = keep_ref[...] * S


def _to_chunked(a):
    return a.reshape(_CHUNKS, _R, _L).transpose(1, 0, 2).reshape(_R, _C)


def kernel(boxes, scores):
    n = scores.shape[0]
    order = jnp.argsort(-scores)
    b = boxes[order]
    s = scores[order]
    area = (b[:, 2] - b[:, 0]) * (b[:, 3] - b[:, 1])
    pad = _NP - n
    cols = [b[:, 0], b[:, 1], b[:, 2], b[:, 3], area, s]
    bt = jnp.concatenate(
        [_to_chunked(jnp.pad(c, (0, pad))) for c in cols], axis=0)
    bs = jnp.pad(jnp.stack(cols[:5], axis=1), ((0, pad), (0, 0))).reshape(-1)

    kept = pl.pallas_call(
        _nms_body,
        in_specs=[pl.BlockSpec(memory_space=pltpu.VMEM),
                  pl.BlockSpec(memory_space=pltpu.SMEM)],
        out_shape=jax.ShapeDtypeStruct((_R, _C), jnp.float32),
        scratch_shapes=[pltpu.VMEM((_R, _C), jnp.float32)],
    )(bt, bs)

    kept = kept.reshape(_R, _CHUNKS, _L).transpose(1, 0, 2).reshape(-1)[:n]
    return jnp.zeros_like(scores).at[order].set(kept)
